# Initial kernel scaffold; baseline (speedup 1.0000x reference)
import functools

import jax
import jax.numpy as jnp
from jax import lax
from jax.experimental import pallas as pl
from jax.experimental.pallas import tpu as pltpu
from jax.experimental.pallas import tpu_sc as plsc

N, D, G = 320000, 128, 1024
NCORES, NSUB = 2, 16
NW = NCORES * NSUB
C = N // NW
R = 400
NBLK = C // R
NGRP = R // 16
ACC_W = D + 16
B1 = 2000


def _scores_body(x_ref, q_ref, e_ref):
    s = jnp.dot(x_ref[...], q_ref[...], preferred_element_type=jnp.float32)
    e_ref[...] = jnp.exp(s)


def _scores(x, query):
    return pl.pallas_call(
        _scores_body,
        grid=(N // B1,),
        in_specs=[pl.BlockSpec((B1, D), lambda i: (i, 0)),
                  pl.BlockSpec((D, 1), lambda i: (0, 0))],
        out_specs=pl.BlockSpec((B1, 1), lambda i: (i, 0)),
        out_shape=jax.ShapeDtypeStruct((N, 1), jnp.float32),
    )(x, query.reshape(D, 1))


def _sc_body(x_hbm, e_hbm, b_hbm, acc_hbm, shared, xb0, xb1, e_buf, b_buf,
             exp_buf, stage_row, idx_ref, sem0, sem1):
    lane = lax.iota(jnp.int32, 16)
    zf = jnp.zeros((16,), jnp.float32)
    cid = lax.axis_index("c")
    sid = lax.axis_index("s")
    wid = cid * NSUB + sid
    base = wid * C

    def _bcast(vec, idx16):
        return jnp.take(vec, idx16, mode="promise_in_bounds")

    def _flush(g, den_vec, accs):
        for k in range(8):
            stage_row[0, pl.ds(16 * k, 16)] = accs[k]
        den_tot = jnp.sum(den_vec)
        stage_row[0, pl.ds(128, 16)] = zf + den_tot
        plsc.store_scatter(idx_ref, [jnp.zeros((16,), jnp.int32)],
                           jnp.zeros((16,), jnp.int32) + g, mask=lane == 0)
        pltpu.sync_copy(stage_row, shared.at[idx_ref], add=True)

    def _accum_row(xb, row, e_bc, accs):
        return [accs[k] + e_bc * xb[row, pl.ds(16 * k, 16)] for k in range(8)]

    def _make_group_body(xb, pos0):
        def body(grp, carry):
            pos = pos0 + grp * 16
            b16 = b_buf[pl.ds(pos, 16)]
            e16 = e_buf[pl.ds(pos, 16)]
            uniform = jnp.all(b16 == carry[0])

            def fast(op):
                g_, den_ = op[0], op[1] + e16
                accs_ = list(op[2:])
                for r in range(16):
                    e_bc = _bcast(e16, jnp.full((16,), r, jnp.int32))
                    accs_ = _accum_row(xb, grp * 16 + r, e_bc, accs_)
                return (g_, den_, *accs_)

            def slow(op):
                def row_body(r, op2):
                    sel = lane == r
                    b_r = jnp.sum(jnp.where(sel, b16, 0))

                    def do_flush(op3):
                        _flush(op3[0], op3[1], list(op3[2:]))
                        return (b_r, zf, *([zf] * 8))

                    op2b = lax.cond(b_r != op2[0], do_flush, lambda o: o, op2)
                    den2 = op2b[1] + jnp.where(sel, e16, 0.0)
                    e_bc = _bcast(e16, jnp.zeros((16,), jnp.int32) + r)
                    accs2 = _accum_row(xb, grp * 16 + r, e_bc, list(op2b[2:]))
                    return (op2b[0], den2, *accs2)

                return lax.fori_loop(0, 16, row_body, op)

            return lax.cond(uniform, fast, slow, carry)
        return body

    pltpu.sync_copy(e_hbm.at[pl.ds(base, C)], e_buf)
    pltpu.sync_copy(b_hbm.at[pl.ds(base, C)], b_buf)

    for i in range(16):
        for k in range(9):
            exp_buf[i, pl.ds(16 * k, 16)] = zf
    for j in range(4):
        pltpu.sync_copy(exp_buf, shared.at[pl.ds(64 * sid + 16 * j, 16)])
    plsc.subcore_barrier()

    b16_0 = b_buf[pl.ds(0, 16)]
    g0 = jnp.sum(jnp.where(lane == 0, b16_0, 0))
    carry = (g0, zf, *([zf] * 8))

    def start(blk, buf, sem):
        pltpu.async_copy(x_hbm.at[pl.ds(base + blk * R, R)], buf, sem)

    def wait(blk, buf, sem):
        pltpu.make_async_copy(
            x_hbm.at[pl.ds(base + blk * R, R)], buf, sem).wait()

    start(0, xb0, sem0)

    def bb_body(bb, carry):
        b_even = 2 * bb
        start(b_even + 1, xb1, sem1)
        wait(b_even, xb0, sem0)
        carry = lax.fori_loop(0, NGRP, _make_group_body(xb0, b_even * R),
                              carry)
        start(b_even + 2, xb0, sem0)
        wait(b_even + 1, xb1, sem1)
        carry = lax.fori_loop(0, NGRP,
                              _make_group_body(xb1, (b_even + 1) * R), carry)
        return carry

    carry = lax.fori_loop(0, (NBLK - 1) // 2, bb_body, carry)
    wait(NBLK - 1, xb0, sem0)
    carry = lax.fori_loop(0, NGRP, _make_group_body(xb0, (NBLK - 1) * R),
                          carry)
    _flush(carry[0], carry[1], list(carry[2:]))

    plsc.subcore_barrier()

    for j in range(4):
        pltpu.sync_copy(shared.at[pl.ds(64 * sid + 16 * j, 16)], exp_buf)
        pltpu.sync_copy(exp_buf, acc_hbm.at[cid, pl.ds(64 * sid + 16 * j, 16)])


def _sc_pool(x, e, b32):
    mesh = plsc.VectorSubcoreMesh(core_axis_name="c", subcore_axis_name="s")
    kern = pl.kernel(
        _sc_body,
        mesh=mesh,
        out_type=jax.ShapeDtypeStruct((NCORES, G, ACC_W), jnp.float32),
        scratch_types=[
            pltpu.VMEM_SHARED((G, ACC_W), jnp.float32),
            pltpu.VMEM((R, D), jnp.float32),
            pltpu.VMEM((R, D), jnp.float32),
            pltpu.VMEM((C,), jnp.float32),
            pltpu.VMEM((C,), jnp.int32),
            pltpu.VMEM((16, ACC_W), jnp.float32),
            pltpu.VMEM((1, ACC_W), jnp.float32),
            pltpu.VMEM((1,), jnp.int32),
            pltpu.SemaphoreType.DMA,
            pltpu.SemaphoreType.DMA,
        ],
    )
    return kern(x, e, b32)


def _combine_body(acc_ref, out_ref):
    num = acc_ref[0, :, 0:128] + acc_ref[1, :, 0:128]
    den = acc_ref[0, :, 128:129] + acc_ref[1, :, 128:129]
    out_ref[...] = jnp.where(den > 0.0, num / den, 0.0)


def _combine(acc):
    return pl.pallas_call(
        _combine_body,
        out_shape=jax.ShapeDtypeStruct((G, D), jnp.float32),
    )(acc)


def kernel(x, batch, query):
    e = _scores(x, query).reshape(N)
    acc = _sc_pool(x, e, batch.astype(jnp.int32))
    return _combine(acc)


# speedup vs baseline: 6.0510x; 6.0510x over previous
import functools

import jax
import jax.numpy as jnp
from jax import lax
from jax.experimental import pallas as pl
from jax.experimental.pallas import tpu as pltpu
from jax.experimental.pallas import tpu_sc as plsc

N, D, G = 320000, 128, 1024
NCORES, NSUB = 2, 16
NW = NCORES * NSUB
C = N // NW
R = 400
NBLK = C // R
NGRP = R // 16
ACC_W = D + 16
B1 = 2000


def _scores_body(x_ref, q_ref, e_ref):
    s = jnp.dot(x_ref[...], q_ref[...], preferred_element_type=jnp.float32)
    e_ref[...] = jnp.exp(s)


def _scores(x, query):
    return pl.pallas_call(
        _scores_body,
        grid=(N // B1,),
        in_specs=[pl.BlockSpec((B1, D), lambda i: (i, 0)),
                  pl.BlockSpec((D, 1), lambda i: (0, 0))],
        out_specs=pl.BlockSpec((B1, 1), lambda i: (i, 0)),
        out_shape=jax.ShapeDtypeStruct((N, 1), jnp.float32),
    )(x, query.reshape(D, 1))


def _sc_body(x_hbm, e_hbm, b_hbm, gprev_hbm, out_hbm, part_hbm,
             xb0, xb1, e_buf, b_buf, gp_buf, srow, zrow, sem0, sem1):
    lane = lax.iota(jnp.int32, 16)
    zf = jnp.zeros((16,), jnp.float32)
    zi = jnp.zeros((16,), jnp.int32)
    cid = lax.axis_index("c")
    sid = lax.axis_index("s")
    wid = cid * NSUB + sid
    base = wid * C

    def _bcast(vec, idx16):
        dnums = lax.GatherDimensionNumbers(
            offset_dims=(), collapsed_slice_dims=(0,), start_index_map=(0,))
        return lax.gather(vec, idx16[:, None], dnums, (1,),
                          mode=lax.GatherScatterMode.PROMISE_IN_BOUNDS)

    def _allsum(v):
        for shift in (8, 4, 2, 1):
            v = v + _bcast(v, jnp.bitwise_xor(lane, shift))
        return v

    pltpu.sync_copy(e_hbm.at[pl.ds(base, C)], e_buf)
    pltpu.sync_copy(b_hbm.at[pl.ds(base, C)], b_buf.at[pl.ds(0, C)])
    pltpu.sync_copy(gprev_hbm, gp_buf.at[pl.ds(0, NW)])

    g_prev = gp_buf[pl.ds(wid, 16)][0]
    g_first = b_buf[pl.ds(0, 16)][0]
    g_last = b_buf[pl.ds(C - 16, 16)][15]

    for k in range(9):
        zrow[pl.ds(16 * k, 16)] = zf

    def _zero_row(gid):
        pltpu.sync_copy(zrow, out_hbm.at[pl.ds(gid * ACC_W, ACC_W)])

    lax.fori_loop(0, g_first - g_prev,
                  lambda i, c: (_zero_row(g_prev + 1 + i), c)[1], 0)

    @pl.when(wid == NW - 1)
    def _():
        lax.fori_loop(0, G - 1 - g_last,
                      lambda i, c: (_zero_row(g_last + 1 + i), c)[1], 0)

    def _store_srow(den_vec, accs):
        for k in range(8):
            srow[pl.ds(16 * k, 16)] = accs[k]
        srow[pl.ds(128, 16)] = _allsum(den_vec)

    def _emit(g_s, den_vec, accs):
        _store_srow(den_vec, accs)

        @pl.when(g_s == g_first)
        def _():
            pltpu.sync_copy(srow, part_hbm.at[pl.ds(wid * ACC_W, ACC_W)])

        @pl.when(g_s != g_first)
        def _():
            pltpu.sync_copy(srow, out_hbm.at[pl.ds(g_s * ACC_W, ACC_W)])

    def _make_group_body(xb, pos0):
        def body(grp, carry):
            pos = pos0 + grp * 16
            e16 = e_buf[pl.ds(pos, 16)]

            def row_body(r, op):
                g, den = op[0], op[1]
                accs = list(op[2:])
                b_r = b_buf[pl.ds(pos + r, 16)][0]
                e_bc = _bcast(e16, zi + r)
                pred = b_r != g

                @pl.when(pred)
                def _():
                    _emit(g, den, accs)
                    lax.fori_loop(0, b_r - g - 1,
                                  lambda i, c: (_zero_row(g + 1 + i), c)[1], 0)

                g2 = jnp.where(pred, b_r, g)
                den2 = (jnp.where(pred, 0.0, den)
                        + jnp.where(lane == r, e16, 0.0))
                row = grp * 16 + r
                accs2 = [jnp.where(pred, 0.0, accs[k])
                         + e_bc * xb[row, pl.ds(16 * k, 16)]
                         for k in range(8)]
                return (g2, den2, *accs2)

            return lax.fori_loop(0, 16, row_body, carry)
        return body

    carry = (g_first, zf, *([zf] * 8))

    def start(blk, buf, sem):
        pltpu.async_copy(x_hbm.at[pl.ds(base + blk * R, R)], buf, sem)

    def wait(blk, buf, sem):
        pltpu.make_async_copy(
            x_hbm.at[pl.ds(base + blk * R, R)], buf, sem).wait()

    start(0, xb0, sem0)

    def bb_body(bb, carry):
        b_even = 2 * bb
        start(b_even + 1, xb1, sem1)
        wait(b_even, xb0, sem0)
        carry = lax.fori_loop(0, NGRP, _make_group_body(xb0, b_even * R),
                              carry)
        start(b_even + 2, xb0, sem0)
        wait(b_even + 1, xb1, sem1)
        carry = lax.fori_loop(0, NGRP,
                              _make_group_body(xb1, (b_even + 1) * R), carry)
        return carry

    carry = lax.fori_loop(0, (NBLK - 1) // 2, bb_body, carry)
    wait(NBLK - 1, xb0, sem0)
    carry = lax.fori_loop(0, NGRP, _make_group_body(xb0, (NBLK - 1) * R),
                          carry)
    _emit(carry[0], carry[1], list(carry[2:]))


def _sc_pool(x, e, b32, gprev):
    mesh = plsc.VectorSubcoreMesh(core_axis_name="c", subcore_axis_name="s")
    kern = pl.kernel(
        _sc_body,
        mesh=mesh,
        out_type=[jax.ShapeDtypeStruct((G * ACC_W,), jnp.float32),
                  jax.ShapeDtypeStruct((NW * ACC_W,), jnp.float32)],
        scratch_types=[
            pltpu.VMEM((R, D), jnp.float32),
            pltpu.VMEM((R, D), jnp.float32),
            pltpu.VMEM((C,), jnp.float32),
            pltpu.VMEM((C + 16,), jnp.int32),
            pltpu.VMEM((NW + 16,), jnp.int32),
            pltpu.VMEM((ACC_W,), jnp.float32),
            pltpu.VMEM((ACC_W,), jnp.float32),
            pltpu.SemaphoreType.DMA,
            pltpu.SemaphoreType.DMA,
        ],
    )
    return kern(x, e, b32, gprev)


def _combine_body(gf_ref, acc_ref, part_ref, out_ref, acc2):
    acc2[...] = acc_ref[...]
    for w in range(NW):
        gw = gf_ref[w]
        acc2[pl.ds(gw, 1), :] = (acc2[pl.ds(gw, 1), :]
                                 + part_ref[pl.ds(w, 1), :])
    num = acc2[:, 0:128]
    den = acc2[:, 128:129]
    out_ref[...] = jnp.where(den > 0.0, num / den, 0.0)


def _combine(gfirst, acc, part):
    return pl.pallas_call(
        _combine_body,
        in_specs=[pl.BlockSpec(memory_space=pltpu.SMEM),
                  pl.BlockSpec((G, ACC_W), lambda: (0, 0)),
                  pl.BlockSpec((NW, ACC_W), lambda: (0, 0))],
        out_specs=pl.BlockSpec((G, D), lambda: (0, 0)),
        out_shape=jax.ShapeDtypeStruct((G, D), jnp.float32),
        scratch_shapes=[pltpu.VMEM((G, ACC_W), jnp.float32)],
    )(gfirst, acc, part)


def kernel(x, batch, query):
    b32 = batch.astype(jnp.int32)
    e = _scores(x, query).reshape(N)
    gfirst = b32[::C]
    gprev = jnp.concatenate([jnp.full((1,), -1, jnp.int32),
                             b32[C - 1::C][:NW - 1]])
    out_flat, part_flat = _sc_pool(x, e, b32, gprev)
    return _combine(gfirst, out_flat.reshape(G, ACC_W),
                    part_flat.reshape(NW, ACC_W))


# speedup vs baseline: 12.4853x; 2.0633x over previous
import functools

import jax
import jax.numpy as jnp
from jax import lax
from jax.experimental import pallas as pl
from jax.experimental.pallas import tpu as pltpu
from jax.experimental.pallas import tpu_sc as plsc

N, D, G = 320000, 128, 1024
NCORES, NSUB = 2, 16
NW = NCORES * NSUB
C = N // NW
R = 400
NBLK = C // R
NGRP = R // 16
ACC_W = D + 16
B1 = 2000


def _scores_body(x_ref, q_ref, e_ref):
    s = jnp.dot(x_ref[...], q_ref[...], preferred_element_type=jnp.float32)
    e_ref[...] = jnp.exp(s)


def _scores(x, query):
    return pl.pallas_call(
        _scores_body,
        grid=(N // B1,),
        in_specs=[pl.BlockSpec((B1, D), lambda i: (i, 0)),
                  pl.BlockSpec((D, 1), lambda i: (0, 0))],
        out_specs=pl.BlockSpec((B1, 1), lambda i: (i, 0)),
        out_shape=jax.ShapeDtypeStruct((N, 1), jnp.float32),
    )(x, query.reshape(D, 1))


def _sc_body(x_hbm, e_hbm, b_hbm, gprev_hbm, out_hbm, part_hbm,
             xb0, xb1, e_buf, b_buf, gp_buf, srow, zrow,
             g_st, den_st, acc_st, sem0, sem1):
    lane = lax.iota(jnp.int32, 16)
    zf = jnp.zeros((16,), jnp.float32)
    zi = jnp.zeros((16,), jnp.int32)
    cid = lax.axis_index("c")
    sid = lax.axis_index("s")
    wid = cid * NSUB + sid
    base = wid * C

    def _bcast(vec, idx16):
        dnums = lax.GatherDimensionNumbers(
            offset_dims=(), collapsed_slice_dims=(0,), start_index_map=(0,))
        return lax.gather(vec, idx16[:, None], dnums, (1,),
                          mode=lax.GatherScatterMode.PROMISE_IN_BOUNDS)

    def _allsum(v):
        for shift in (8, 4, 2, 1):
            v = v + _bcast(v, jnp.bitwise_xor(lane, shift))
        return v

    pltpu.sync_copy(e_hbm.at[pl.ds(base, C)], e_buf)
    pltpu.sync_copy(b_hbm.at[pl.ds(base, C)], b_buf.at[pl.ds(0, C)])
    pltpu.sync_copy(gprev_hbm, gp_buf.at[pl.ds(0, NW)])

    g_prev = gp_buf[pl.ds(wid, 16)][0]
    g_first = b_buf[pl.ds(0, 16)][0]
    g_last = b_buf[pl.ds(C - 16, 16)][15]

    for k in range(9):
        zrow[pl.ds(16 * k, 16)] = zf

    def _zero_row(gid):
        pltpu.sync_copy(zrow, out_hbm.at[pl.ds(gid * ACC_W, ACC_W)])

    lax.fori_loop(0, g_first - g_prev,
                  lambda i, c: (_zero_row(g_prev + 1 + i), c)[1], 0)

    @pl.when(wid == NW - 1)
    def _():
        lax.fori_loop(0, G - 1 - g_last,
                      lambda i, c: (_zero_row(g_last + 1 + i), c)[1], 0)

    def _store_srow(den_vec, accs):
        for k in range(8):
            srow[pl.ds(16 * k, 16)] = accs[k]
        srow[pl.ds(128, 16)] = _allsum(den_vec)

    def _emit(g_s, den_vec, accs):
        _store_srow(den_vec, accs)

        @pl.when(g_s == g_first)
        def _():
            pltpu.sync_copy(srow, part_hbm.at[pl.ds(wid * ACC_W, ACC_W)])

        @pl.when(g_s != g_first)
        def _():
            pltpu.sync_copy(srow, out_hbm.at[pl.ds(g_s * ACC_W, ACC_W)])

    def _load_accs():
        return [acc_st[pl.ds(16 * k, 16)] for k in range(8)]

    def _make_group_body(xb, pos0):
        def body(grp, carry):
            pos = pos0 + grp * 16
            e16 = e_buf[pl.ds(pos, 16)]
            g = g_st[...][0]
            uniform = b_buf[pl.ds(pos + 15, 16)][0] == g

            @pl.when(uniform)
            def _():
                accs = _load_accs()
                for r in range(16):
                    e_r = e16[r]
                    row = grp * 16 + r
                    accs = [accs[k] + e_r * xb[row, pl.ds(16 * k, 16)]
                            for k in range(8)]
                for k in range(8):
                    acc_st[pl.ds(16 * k, 16)] = accs[k]
                den_st[...] = den_st[...] + e16

            @pl.when(jnp.logical_not(uniform))
            def _():
                def row_body(r, c):
                    b_r = b_buf[pl.ds(pos + r, 16)][0]
                    g_c = g_st[...][0]
                    pred = b_r != g_c

                    @pl.when(pred)
                    def _():
                        _emit(g_c, den_st[...], _load_accs())
                        lax.fori_loop(0, b_r - g_c - 1,
                                      lambda i, cc:
                                      (_zero_row(g_c + 1 + i), cc)[1], 0)
                        for k in range(8):
                            acc_st[pl.ds(16 * k, 16)] = zf
                        den_st[...] = zf
                        g_st[...] = zi + b_r

                    e_bc = _bcast(e16, zi + r)
                    row = grp * 16 + r
                    for k in range(8):
                        acc_st[pl.ds(16 * k, 16)] = (
                            acc_st[pl.ds(16 * k, 16)]
                            + e_bc * xb[row, pl.ds(16 * k, 16)])
                    den_st[...] = den_st[...] + jnp.where(lane == r, e16, 0.0)
                    return c

                lax.fori_loop(0, 16, row_body, 0)

            return carry
        return body

    g_st[...] = zi + g_first
    den_st[...] = zf
    for k in range(8):
        acc_st[pl.ds(16 * k, 16)] = zf
    carry = 0

    def start(blk, buf, sem):
        pltpu.async_copy(x_hbm.at[pl.ds(base + blk * R, R)], buf, sem)

    def wait(blk, buf, sem):
        pltpu.make_async_copy(
            x_hbm.at[pl.ds(base + blk * R, R)], buf, sem).wait()

    start(0, xb0, sem0)

    def bb_body(bb, carry):
        b_even = 2 * bb
        start(b_even + 1, xb1, sem1)
        wait(b_even, xb0, sem0)
        carry = lax.fori_loop(0, NGRP, _make_group_body(xb0, b_even * R),
                              carry)
        start(b_even + 2, xb0, sem0)
        wait(b_even + 1, xb1, sem1)
        carry = lax.fori_loop(0, NGRP,
                              _make_group_body(xb1, (b_even + 1) * R), carry)
        return carry

    carry = lax.fori_loop(0, (NBLK - 1) // 2, bb_body, carry)
    wait(NBLK - 1, xb0, sem0)
    carry = lax.fori_loop(0, NGRP, _make_group_body(xb0, (NBLK - 1) * R),
                          carry)
    _emit(g_st[...][0], den_st[...], _load_accs())


def _sc_pool(x, e, b32, gprev):
    mesh = plsc.VectorSubcoreMesh(core_axis_name="c", subcore_axis_name="s")
    kern = pl.kernel(
        _sc_body,
        mesh=mesh,
        out_type=[jax.ShapeDtypeStruct((G * ACC_W,), jnp.float32),
                  jax.ShapeDtypeStruct((NW * ACC_W,), jnp.float32)],
        scratch_types=[
            pltpu.VMEM((R, D), jnp.float32),
            pltpu.VMEM((R, D), jnp.float32),
            pltpu.VMEM((C,), jnp.float32),
            pltpu.VMEM((C + 16,), jnp.int32),
            pltpu.VMEM((NW + 16,), jnp.int32),
            pltpu.VMEM((ACC_W,), jnp.float32),
            pltpu.VMEM((ACC_W,), jnp.float32),
            pltpu.VMEM((16,), jnp.int32),
            pltpu.VMEM((16,), jnp.float32),
            pltpu.VMEM((D,), jnp.float32),
            pltpu.SemaphoreType.DMA,
            pltpu.SemaphoreType.DMA,
        ],
    )
    return kern(x, e, b32, gprev)


def _combine_body(gf_ref, acc_ref, part_ref, out_ref, acc2):
    acc2[...] = acc_ref[...]
    for w in range(NW):
        gw = gf_ref[w]
        acc2[pl.ds(gw, 1), :] = (acc2[pl.ds(gw, 1), :]
                                 + part_ref[pl.ds(w, 1), :])
    num = acc2[:, 0:128]
    den = acc2[:, 128:129]
    out_ref[...] = jnp.where(den > 0.0, num / den, 0.0)


def _combine(gfirst, acc, part):
    return pl.pallas_call(
        _combine_body,
        in_specs=[pl.BlockSpec(memory_space=pltpu.SMEM),
                  pl.BlockSpec((G, ACC_W), lambda: (0, 0)),
                  pl.BlockSpec((NW, ACC_W), lambda: (0, 0))],
        out_specs=pl.BlockSpec((G, D), lambda: (0, 0)),
        out_shape=jax.ShapeDtypeStruct((G, D), jnp.float32),
        scratch_shapes=[pltpu.VMEM((G, ACC_W), jnp.float32)],
    )(gfirst, acc, part)


def kernel(x, batch, query):
    b32 = batch.astype(jnp.int32)
    e = _scores(x, query).reshape(N)
    gfirst = b32[::C]
    gprev = jnp.concatenate([jnp.full((1,), -1, jnp.int32),
                             b32[C - 1::C][:NW - 1]])
    out_flat, part_flat = _sc_pool(x, e, b32, gprev)
    return _combine(gfirst, out_flat.reshape(G, ACC_W),
                    part_flat.reshape(NW, ACC_W))


# speedup vs baseline: 26.2445x; 2.1020x over previous
import jax
import jax.numpy as jnp
from jax import lax
from jax.experimental import pallas as pl
from jax.experimental.pallas import tpu as pltpu
from jax.experimental.pallas import tpu_sc as plsc

N, D, G = 320000, 128, 1024
NCORES, NSUB = 2, 16
NW = NCORES * NSUB
C = N // NW
R = 400
NBLK = C // R
NGRP = R // 16
ACC_W = D + 16


def _sc_body(x_hbm, b_hbm, q_hbm, gprev_hbm, out_hbm, part_hbm,
             xb0, xb1, b_buf, q_buf, gp_buf, srow, zrow,
             g_st, den_st, acc_st, sem0, sem1):
    lane = lax.iota(jnp.int32, 16)
    zf = jnp.zeros((16,), jnp.float32)
    zi = jnp.zeros((16,), jnp.int32)
    cid = lax.axis_index("c")
    sid = lax.axis_index("s")
    wid = cid * NSUB + sid
    base = wid * C

    def _bcast(vec, idx16):
        dnums = lax.GatherDimensionNumbers(
            offset_dims=(), collapsed_slice_dims=(0,), start_index_map=(0,))
        return lax.gather(vec, idx16[:, None], dnums, (1,),
                          mode=lax.GatherScatterMode.PROMISE_IN_BOUNDS)

    def _allsum(v):
        for shift in (8, 4, 2, 1):
            v = v + _bcast(v, jnp.bitwise_xor(lane, shift))
        return v

    pltpu.sync_copy(b_hbm.at[pl.ds(base, C)], b_buf.at[pl.ds(0, C)])
    pltpu.sync_copy(q_hbm, q_buf)
    pltpu.sync_copy(gprev_hbm, gp_buf.at[pl.ds(0, NW)])

    g_prev = gp_buf[pl.ds(wid, 16)][0]
    g_first = b_buf[pl.ds(0, 16)][0]
    g_last = b_buf[pl.ds(C - 16, 16)][15]

    for k in range(9):
        zrow[pl.ds(16 * k, 16)] = zf

    def _zero_row(gid):
        pltpu.sync_copy(zrow, out_hbm.at[pl.ds(gid * ACC_W, ACC_W)])

    lax.fori_loop(0, g_first - g_prev,
                  lambda i, c: (_zero_row(g_prev + 1 + i), c)[1], 0)

    @pl.when(wid == NW - 1)
    def _():
        lax.fori_loop(0, G - 1 - g_last,
                      lambda i, c: (_zero_row(g_last + 1 + i), c)[1], 0)

    def _load_accs():
        return [acc_st[pl.ds(16 * k, 16)] for k in range(8)]

    def _emit(g_s, den_vec, accs):
        for k in range(8):
            srow[pl.ds(16 * k, 16)] = accs[k]
        srow[pl.ds(128, 16)] = den_vec

        @pl.when(g_s == g_first)
        def _():
            pltpu.sync_copy(srow, part_hbm.at[pl.ds(wid * ACC_W, ACC_W)])

        @pl.when(g_s != g_first)
        def _():
            pltpu.sync_copy(srow, out_hbm.at[pl.ds(g_s * ACC_W, ACC_W)])

    def _make_group_body(xb, pos0):
        def body(grp, carry):
            pos = pos0 + grp * 16
            qk = [q_buf[pl.ds(16 * k, 16)] for k in range(8)]
            g = g_st[...][0]
            uniform = b_buf[pl.ds(pos + 15, 16)][0] == g

            def _row_e(row):
                xk = [xb[row, pl.ds(16 * k, 16)] for k in range(8)]
                p = xk[0] * qk[0]
                for k in range(1, 8):
                    p = p + xk[k] * qk[k]
                return xk, jnp.exp(_allsum(p))

            @pl.when(uniform)
            def _():
                accs = _load_accs()
                den = den_st[...]
                for r in range(16):
                    xk, e_bc = _row_e(grp * 16 + r)
                    accs = [accs[k] + e_bc * xk[k] for k in range(8)]
                    den = den + e_bc
                for k in range(8):
                    acc_st[pl.ds(16 * k, 16)] = accs[k]
                den_st[...] = den

            @pl.when(jnp.logical_not(uniform))
            def _():
                def row_body(r, c):
                    b_r = b_buf[pl.ds(pos + r, 16)][0]
                    g_c = g_st[...][0]
                    pred = b_r != g_c

                    @pl.when(pred)
                    def _():
                        _emit(g_c, den_st[...], _load_accs())
                        lax.fori_loop(0, b_r - g_c - 1,
                                      lambda i, cc:
                                      (_zero_row(g_c + 1 + i), cc)[1], 0)
                        for k in range(8):
                            acc_st[pl.ds(16 * k, 16)] = zf
                        den_st[...] = zf
                        g_st[...] = zi + b_r

                    xk, e_bc = _row_e(grp * 16 + r)
                    for k in range(8):
                        acc_st[pl.ds(16 * k, 16)] = (
                            acc_st[pl.ds(16 * k, 16)] + e_bc * xk[k])
                    den_st[...] = den_st[...] + e_bc
                    return c

                lax.fori_loop(0, 16, row_body, 0)

            return carry
        return body

    g_st[...] = zi + g_first
    den_st[...] = zf
    for k in range(8):
        acc_st[pl.ds(16 * k, 16)] = zf
    carry = 0

    def start(blk, buf, sem):
        pltpu.async_copy(x_hbm.at[pl.ds(base + blk * R, R)], buf, sem)

    def wait(blk, buf, sem):
        pltpu.make_async_copy(
            x_hbm.at[pl.ds(base + blk * R, R)], buf, sem).wait()

    start(0, xb0, sem0)

    def bb_body(bb, carry):
        b_even = 2 * bb
        start(b_even + 1, xb1, sem1)
        wait(b_even, xb0, sem0)
        carry = lax.fori_loop(0, NGRP, _make_group_body(xb0, b_even * R),
                              carry)
        start(b_even + 2, xb0, sem0)
        wait(b_even + 1, xb1, sem1)
        carry = lax.fori_loop(0, NGRP,
                              _make_group_body(xb1, (b_even + 1) * R), carry)
        return carry

    carry = lax.fori_loop(0, (NBLK - 1) // 2, bb_body, carry)
    wait(NBLK - 1, xb0, sem0)
    carry = lax.fori_loop(0, NGRP, _make_group_body(xb0, (NBLK - 1) * R),
                          carry)
    _emit(g_st[...][0], den_st[...], _load_accs())


def _sc_pool(x, b32, query, gprev):
    mesh = plsc.VectorSubcoreMesh(core_axis_name="c", subcore_axis_name="s")
    kern = pl.kernel(
        _sc_body,
        mesh=mesh,
        out_type=[jax.ShapeDtypeStruct((G * ACC_W,), jnp.float32),
                  jax.ShapeDtypeStruct((NW * ACC_W,), jnp.float32)],
        scratch_types=[
            pltpu.VMEM((R, D), jnp.float32),
            pltpu.VMEM((R, D), jnp.float32),
            pltpu.VMEM((C + 16,), jnp.int32),
            pltpu.VMEM((D,), jnp.float32),
            pltpu.VMEM((NW + 16,), jnp.int32),
            pltpu.VMEM((ACC_W,), jnp.float32),
            pltpu.VMEM((ACC_W,), jnp.float32),
            pltpu.VMEM((16,), jnp.int32),
            pltpu.VMEM((16,), jnp.float32),
            pltpu.VMEM((D,), jnp.float32),
            pltpu.SemaphoreType.DMA,
            pltpu.SemaphoreType.DMA,
        ],
    )
    return kern(x, b32, query, gprev)


def _combine_body(gf_ref, acc_ref, part_ref, out_ref, acc2):
    acc2[...] = acc_ref[...]
    for w in range(NW):
        gw = gf_ref[w]
        acc2[pl.ds(gw, 1), :] = (acc2[pl.ds(gw, 1), :]
                                 + part_ref[pl.ds(w, 1), :])
    num = acc2[:, 0:128]
    den = acc2[:, 128:129]
    out_ref[...] = jnp.where(den > 0.0, num / den, 0.0)


def _combine(gfirst, acc, part):
    return pl.pallas_call(
        _combine_body,
        in_specs=[pl.BlockSpec(memory_space=pltpu.SMEM),
                  pl.BlockSpec((G, ACC_W), lambda: (0, 0)),
                  pl.BlockSpec((NW, ACC_W), lambda: (0, 0))],
        out_specs=pl.BlockSpec((G, D), lambda: (0, 0)),
        out_shape=jax.ShapeDtypeStruct((G, D), jnp.float32),
        scratch_shapes=[pltpu.VMEM((G, ACC_W), jnp.float32)],
    )(gfirst, acc, part)


def kernel(x, batch, query):
    b32 = batch.astype(jnp.int32)
    gfirst = b32[::C]
    gprev = jnp.concatenate([jnp.full((1,), -1, jnp.int32),
                             b32[C - 1::C][:NW - 1]])
    out_flat, part_flat = _sc_pool(x, b32, query, gprev)
    return _combine(gfirst, out_flat.reshape(G, ACC_W),
                    part_flat.reshape(NW, ACC_W))


# speedup vs baseline: 26.2624x; 1.0007x over previous
import jax
import jax.numpy as jnp
from jax import lax
from jax.experimental import pallas as pl
from jax.experimental.pallas import tpu as pltpu
from jax.experimental.pallas import tpu_sc as plsc

N, D, G = 320000, 128, 1024
NCORES, NSUB = 2, 16
NW = NCORES * NSUB
C = N // NW
R = 400
NBLK = C // R
NGRP = R // 16
ACC_W = D + 16


def _sc_body(x_hbm, b_hbm, q_hbm, out_hbm, part_hbm,
             xb0, xb1, b_buf, q_buf, bp_buf, srow, zrow,
             g_st, den_st, acc_st, sem0, sem1):
    lane = lax.iota(jnp.int32, 16)
    zf = jnp.zeros((16,), jnp.float32)
    zi = jnp.zeros((16,), jnp.int32)
    cid = lax.axis_index("c")
    sid = lax.axis_index("s")
    wid = cid * NSUB + sid
    base = wid * C

    def _bcast(vec, idx16):
        dnums = lax.GatherDimensionNumbers(
            offset_dims=(), collapsed_slice_dims=(0,), start_index_map=(0,))
        return lax.gather(vec, idx16[:, None], dnums, (1,),
                          mode=lax.GatherScatterMode.PROMISE_IN_BOUNDS)

    def _allsum(v):
        for shift in (8, 4, 2, 1):
            v = v + _bcast(v, jnp.bitwise_xor(lane, shift))
        return v

    pltpu.sync_copy(b_hbm.at[pl.ds(base, C)], b_buf.at[pl.ds(0, C)])
    pltpu.sync_copy(q_hbm, q_buf)
    bp_off = pl.multiple_of(jnp.maximum(base - 16, 0), 8)
    pltpu.sync_copy(b_hbm.at[pl.ds(bp_off, 16)], bp_buf)

    g_prev = jnp.where(wid == 0, -1, bp_buf[...][15])
    g_first = b_buf[pl.ds(0, 16)][0]
    g_last = b_buf[pl.ds(C - 16, 16)][15]

    for k in range(9):
        zrow[pl.ds(16 * k, 16)] = zf

    def _zero_row(gid):
        pltpu.sync_copy(zrow, out_hbm.at[pl.ds(gid * ACC_W, ACC_W)])

    lax.fori_loop(0, g_first - g_prev,
                  lambda i, c: (_zero_row(g_prev + 1 + i), c)[1], 0)

    @pl.when(wid == NW - 1)
    def _():
        lax.fori_loop(0, G - 1 - g_last,
                      lambda i, c: (_zero_row(g_last + 1 + i), c)[1], 0)

    def _load_accs():
        return [acc_st[pl.ds(16 * k, 16)] for k in range(8)]

    def _emit(g_s, den_vec, accs):
        for k in range(8):
            srow[pl.ds(16 * k, 16)] = accs[k]
        srow[pl.ds(128, 16)] = den_vec

        @pl.when(g_s == g_first)
        def _():
            pltpu.sync_copy(srow, part_hbm.at[pl.ds(wid * ACC_W, ACC_W)])

        @pl.when(g_s != g_first)
        def _():
            pltpu.sync_copy(srow, out_hbm.at[pl.ds(g_s * ACC_W, ACC_W)])

    def _make_group_body(xb, pos0):
        def body(grp, carry):
            pos = pos0 + grp * 16
            qk = [q_buf[pl.ds(16 * k, 16)] for k in range(8)]
            g = g_st[...][0]
            uniform = b_buf[pl.ds(pos + 15, 16)][0] == g

            def _row_e(row):
                xk = [xb[row, pl.ds(16 * k, 16)] for k in range(8)]
                p = xk[0] * qk[0]
                for k in range(1, 8):
                    p = p + xk[k] * qk[k]
                return xk, jnp.exp(_allsum(p))

            @pl.when(uniform)
            def _():
                accs = _load_accs()
                den = den_st[...]
                for r in range(16):
                    xk, e_bc = _row_e(grp * 16 + r)
                    accs = [accs[k] + e_bc * xk[k] for k in range(8)]
                    den = den + e_bc
                for k in range(8):
                    acc_st[pl.ds(16 * k, 16)] = accs[k]
                den_st[...] = den

            @pl.when(jnp.logical_not(uniform))
            def _():
                def row_body(r, c):
                    b_r = b_buf[pl.ds(pos + r, 16)][0]
                    g_c = g_st[...][0]
                    pred = b_r != g_c

                    @pl.when(pred)
                    def _():
                        _emit(g_c, den_st[...], _load_accs())
                        lax.fori_loop(0, b_r - g_c - 1,
                                      lambda i, cc:
                                      (_zero_row(g_c + 1 + i), cc)[1], 0)
                        for k in range(8):
                            acc_st[pl.ds(16 * k, 16)] = zf
                        den_st[...] = zf
                        g_st[...] = zi + b_r

                    xk, e_bc = _row_e(grp * 16 + r)
                    for k in range(8):
                        acc_st[pl.ds(16 * k, 16)] = (
                            acc_st[pl.ds(16 * k, 16)] + e_bc * xk[k])
                    den_st[...] = den_st[...] + e_bc
                    return c

                lax.fori_loop(0, 16, row_body, 0)

            return carry
        return body

    g_st[...] = zi + g_first
    den_st[...] = zf
    for k in range(8):
        acc_st[pl.ds(16 * k, 16)] = zf
    carry = 0

    def start(blk, buf, sem):
        pltpu.async_copy(x_hbm.at[pl.ds(base + blk * R, R)], buf, sem)

    def wait(blk, buf, sem):
        pltpu.make_async_copy(
            x_hbm.at[pl.ds(base + blk * R, R)], buf, sem).wait()

    start(0, xb0, sem0)

    def bb_body(bb, carry):
        b_even = 2 * bb
        start(b_even + 1, xb1, sem1)
        wait(b_even, xb0, sem0)
        carry = lax.fori_loop(0, NGRP, _make_group_body(xb0, b_even * R),
                              carry)
        start(b_even + 2, xb0, sem0)
        wait(b_even + 1, xb1, sem1)
        carry = lax.fori_loop(0, NGRP,
                              _make_group_body(xb1, (b_even + 1) * R), carry)
        return carry

    carry = lax.fori_loop(0, (NBLK - 1) // 2, bb_body, carry)
    wait(NBLK - 1, xb0, sem0)
    carry = lax.fori_loop(0, NGRP, _make_group_body(xb0, (NBLK - 1) * R),
                          carry)
    _emit(g_st[...][0], den_st[...], _load_accs())


def _sc_pool(x, b32, query):
    mesh = plsc.VectorSubcoreMesh(core_axis_name="c", subcore_axis_name="s")
    kern = pl.kernel(
        _sc_body,
        mesh=mesh,
        out_type=[jax.ShapeDtypeStruct((G * ACC_W,), jnp.float32),
                  jax.ShapeDtypeStruct((NW * ACC_W,), jnp.float32)],
        scratch_types=[
            pltpu.VMEM((R, D), jnp.float32),
            pltpu.VMEM((R, D), jnp.float32),
            pltpu.VMEM((C + 16,), jnp.int32),
            pltpu.VMEM((D,), jnp.float32),
            pltpu.VMEM((16,), jnp.int32),
            pltpu.VMEM((ACC_W,), jnp.float32),
            pltpu.VMEM((ACC_W,), jnp.float32),
            pltpu.VMEM((16,), jnp.int32),
            pltpu.VMEM((16,), jnp.float32),
            pltpu.VMEM((D,), jnp.float32),
            pltpu.SemaphoreType.DMA,
            pltpu.SemaphoreType.DMA,
        ],
    )
    return kern(x, b32, query)


def _combine_body(gf_ref, acc_ref, part_ref, out_ref, acc2):
    acc2[...] = acc_ref[...]
    for w in range(NW):
        gw = gf_ref[w]
        acc2[pl.ds(gw, 1), :] = (acc2[pl.ds(gw, 1), :]
                                 + part_ref[pl.ds(w, 1), :])
    num = acc2[:, 0:128]
    den = acc2[:, 128:129]
    out_ref[...] = jnp.where(den > 0.0, num / den, 0.0)


def _combine(gfirst, acc, part):
    return pl.pallas_call(
        _combine_body,
        in_specs=[pl.BlockSpec(memory_space=pltpu.SMEM),
                  pl.BlockSpec((G, ACC_W), lambda: (0, 0)),
                  pl.BlockSpec((NW, ACC_W), lambda: (0, 0))],
        out_specs=pl.BlockSpec((G, D), lambda: (0, 0)),
        out_shape=jax.ShapeDtypeStruct((G, D), jnp.float32),
        scratch_shapes=[pltpu.VMEM((G, ACC_W), jnp.float32)],
    )(gfirst, acc, part)


def kernel(x, batch, query):
    b32 = batch.astype(jnp.int32)
    gfirst = b32[::C]
    out_flat, part_flat = _sc_pool(x, b32, query)
    return _combine(gfirst, out_flat.reshape(G, ACC_W),
                    part_flat.reshape(NW, ACC_W))
